# Initial kernel scaffold; baseline (speedup 1.0000x reference)
import functools

import jax
import jax.numpy as jnp
from jax import lax
from jax.experimental import pallas as pl
from jax.experimental.pallas import tpu as pltpu
from jax.experimental.pallas import tpu_sc as plsc

NC, NS, L = 2, 16, 16
NW = NC * NS
KLEV = 32


def _build_sc_call(ntok, h, tn):
    hpt = h // NW
    nchunk = ntok // tn
    ngrp = hpt // L

    def body(x_hbm, tex_hbm, y_hbm, out_hbm,
             tex_v, y_v, xb0, xb1, ob0, ob1,
             sin0, sin1, sout0, sout1):
        wid = lax.axis_index("s") * NC + lax.axis_index("c")
        h0 = wid * hpt
        pltpu.sync_copy(tex_hbm.at[pl.ds(h0, hpt), :], tex_v)
        pltpu.sync_copy(y_hbm.at[pl.ds(h0, hpt), :], y_v)

        def in_copy(c, buf, sem):
            return pltpu.make_async_copy(
                x_hbm.at[pl.ds(c * tn, tn), pl.ds(h0, hpt)], buf, sem)

        def out_copy(c, buf, sem):
            return pltpu.make_async_copy(
                buf, out_hbm.at[pl.ds(c * tn, tn), pl.ds(h0, hpt)], sem)

        in_copy(0, xb0, sin0).start()
        in_copy(1, xb1, sin1).start()

        lane = lax.iota(jnp.int32, L)
        chvs = [lane + L * g for g in range(ngrp)]
        c8 = jnp.full((L,), 8, jnp.int32)
        c16 = jnp.full((L,), 16, jnp.int32)
        c24 = jnp.full((L,), 24, jnp.int32)
        t8s = [plsc.load_gather(tex_v, [chv, c8]) for chv in chvs]
        t16s = [plsc.load_gather(tex_v, [chv, c16]) for chv in chvs]
        t24s = [plsc.load_gather(tex_v, [chv, c24]) for chv in chvs]
        zero = jnp.zeros((L,), jnp.int32)

        def compute(xb, ob):
            def row(r, carry):
                for g in range(ngrp):
                    xv = xb[r, pl.ds(L * g, L)]
                    m1 = t16s[g] < xv
                    idx = jnp.where(m1, c16, zero)
                    tc = jnp.where(m1, t24s[g], t8s[g])
                    idx = jnp.where(tc < xv, idx + 8, idx)
                    for b in (4, 2, 1):
                        m = idx + b
                        t = plsc.load_gather(tex_v, [chvs[g], m])
                        idx = jnp.where(t < xv, m, idx)
                    ob[r, pl.ds(L * g, L)] = plsc.load_gather(y_v, [chvs[g], idx])
                return carry
            lax.fori_loop(0, tn, row, 0)

        def pair(i, carry):
            for p, (xb, ob, sin, sout) in enumerate(
                    ((xb0, ob0, sin0, sout0), (xb1, ob1, sin1, sout1))):
                c = 2 * i + p
                in_copy(c, xb, sin).wait()
                pl.when(i > 0)(lambda ob=ob, sout=sout, c=c:
                               out_copy(c, ob, sout).wait())
                compute(xb, ob)
                out_copy(c, ob, sout).start()
                pl.when(i < nchunk // 2 - 1)(
                    lambda xb=xb, sin=sin, c=c: in_copy(c + 2, xb, sin).start())
            return carry

        lax.fori_loop(0, nchunk // 2, pair, 0)
        out_copy(nchunk - 2, ob0, sout0).wait()
        out_copy(nchunk - 1, ob1, sout1).wait()

    return pl.kernel(
        body,
        out_type=jax.ShapeDtypeStruct((ntok, h), jnp.float32),
        mesh=plsc.VectorSubcoreMesh(
            core_axis_name="c", subcore_axis_name="s",
            num_cores=NC, num_subcores=NS),
        scratch_types=[
            pltpu.VMEM((hpt, KLEV), jnp.float32),
            pltpu.VMEM((hpt, KLEV), jnp.float32),
            pltpu.VMEM((tn, hpt), jnp.float32),
            pltpu.VMEM((tn, hpt), jnp.float32),
            pltpu.VMEM((tn, hpt), jnp.float32),
            pltpu.VMEM((tn, hpt), jnp.float32),
            pltpu.SemaphoreType.DMA,
            pltpu.SemaphoreType.DMA,
            pltpu.SemaphoreType.DMA,
            pltpu.SemaphoreType.DMA,
        ],
    )


@functools.lru_cache(maxsize=None)
def _get_sc_call(ntok, h, tn):
    return _build_sc_call(ntok, h, tn)


def kernel(x, thresholds, y):
    shape = x.shape
    h = shape[-1]
    x2 = x.reshape(-1, h)
    ntok = x2.shape[0]
    tex = jnp.concatenate(
        [jnp.full((h, 1), -jnp.inf, jnp.float32), thresholds], axis=1)
    out = _get_sc_call(ntok, h, 256)(x2, tex, y)
    return out.reshape(shape)


# speedup vs baseline: 1085.9445x; 1085.9445x over previous
import functools

import jax
import jax.numpy as jnp
from jax import lax
from jax.experimental import pallas as pl
from jax.experimental.pallas import tpu as pltpu
from jax.experimental.pallas import tpu_sc as plsc

NC, NS, L = 2, 16, 16
NW = NC * NS
KLEV = 32
CPG = 128
NCG = 16
NTH = NW // NCG


def _build_sc_call(ntok, h, tn):
    tpw = ntok // NTH
    nchunk = tpw // tn
    ngrp = CPG // L

    def body(x_hbm, tab_hbm, out_hbm,
             tab_v, xb0, xb1, ob0, ob1,
             sin0, sin1, sout0, sout1):
        wid = lax.axis_index("s") * NC + lax.axis_index("c")
        cg = lax.rem(wid, NCG)
        th = wid // NCG
        h0 = cg * CPG
        n0 = th * tpw
        pltpu.sync_copy(tab_hbm.at[pl.ds(h0 * 2 * KLEV, CPG * 2 * KLEV)], tab_v)

        def in_copy(c, buf, sem):
            return pltpu.make_async_copy(
                x_hbm.at[pl.ds(n0 + c * tn, tn), pl.ds(h0, CPG)], buf, sem)

        def out_copy(c, buf, sem):
            return pltpu.make_async_copy(
                buf, out_hbm.at[pl.ds(n0 + c * tn, tn), pl.ds(h0, CPG)], sem)

        in_copy(0, xb0, sin0).start()
        in_copy(1, xb1, sin1).start()

        lane = lax.iota(jnp.int32, L)
        chbs = [(lane + L * g) * (2 * KLEV) for g in range(ngrp)]
        c16 = jnp.full((L,), 16, jnp.int32)
        c32 = jnp.full((L,), KLEV, jnp.int32)
        t8s = [plsc.load_gather(tab_v, [chb + 8]) for chb in chbs]
        t16s = [plsc.load_gather(tab_v, [chb + 16]) for chb in chbs]
        t24s = [plsc.load_gather(tab_v, [chb + 24]) for chb in chbs]
        zero = jnp.zeros((L,), jnp.int32)

        def compute(xb, ob):
            def row(r, carry):
                for g in range(ngrp):
                    xv = xb[r, pl.ds(L * g, L)]
                    m1 = t16s[g] < xv
                    idx = jnp.where(m1, c16, zero)
                    tc = jnp.where(m1, t24s[g], t8s[g])
                    idx = jnp.where(tc < xv, idx + 8, idx)
                    for b in (4, 2, 1):
                        m = idx + b
                        t = plsc.load_gather(tab_v, [chbs[g] + m])
                        idx = jnp.where(t < xv, m, idx)
                    ob[r, pl.ds(L * g, L)] = plsc.load_gather(
                        tab_v, [chbs[g] + idx + c32])
                return carry
            lax.fori_loop(0, tn, row, 0)

        def pair(i, carry):
            for p, (xb, ob, sin, sout) in enumerate(
                    ((xb0, ob0, sin0, sout0), (xb1, ob1, sin1, sout1))):
                c = 2 * i + p
                in_copy(c, xb, sin).wait()
                pl.when(i > 0)(lambda ob=ob, sout=sout, c=c:
                               out_copy(c, ob, sout).wait())
                compute(xb, ob)
                out_copy(c, ob, sout).start()
                pl.when(i < nchunk // 2 - 1)(
                    lambda xb=xb, sin=sin, c=c: in_copy(c + 2, xb, sin).start())
            return carry

        lax.fori_loop(0, nchunk // 2, pair, 0)
        out_copy(nchunk - 2, ob0, sout0).wait()
        out_copy(nchunk - 1, ob1, sout1).wait()

    return pl.kernel(
        body,
        compiler_params=pltpu.CompilerParams(needs_layout_passes=False),
        out_type=jax.ShapeDtypeStruct((ntok, h), jnp.float32),
        mesh=plsc.VectorSubcoreMesh(
            core_axis_name="c", subcore_axis_name="s",
            num_cores=NC, num_subcores=NS),
        scratch_types=[
            pltpu.VMEM((CPG * 2 * KLEV,), jnp.float32),
            pltpu.VMEM((tn, CPG), jnp.float32),
            pltpu.VMEM((tn, CPG), jnp.float32),
            pltpu.VMEM((tn, CPG), jnp.float32),
            pltpu.VMEM((tn, CPG), jnp.float32),
            pltpu.SemaphoreType.DMA,
            pltpu.SemaphoreType.DMA,
            pltpu.SemaphoreType.DMA,
            pltpu.SemaphoreType.DMA,
        ],
    )


@functools.lru_cache(maxsize=None)
def _get_sc_call(ntok, h, tn):
    return _build_sc_call(ntok, h, tn)


def kernel(x, thresholds, y):
    shape = x.shape
    h = shape[-1]
    x2 = x.reshape(-1, h)
    ntok = x2.shape[0]
    tab = jnp.concatenate(
        [jnp.full((h, 1), -jnp.inf, jnp.float32), thresholds, y],
        axis=1).reshape(-1)
    out = _get_sc_call(ntok, h, 128)(x2, tab)
    return out.reshape(shape)


# speedup vs baseline: 3810.9424x; 3.5093x over previous
import functools

import jax
import jax.numpy as jnp
from jax import lax
from jax.experimental import pallas as pl
from jax.experimental.pallas import tpu as pltpu
from jax.experimental.pallas import tpu_sc as plsc

NC, NS, L = 2, 16, 16
NW = NC * NS
KLEV = 32
CPG = 128
NCG = 16
NTH = NW // NCG


def _build_sc_call(ntok, h, tn):
    tpw = ntok // NTH
    nchunk = tpw // tn
    ngrp = CPG // L

    def body(x_hbm, tab_hbm, out_hbm,
             tab_v, xb0, xb1, ob0, ob1,
             sin0, sin1, sout0, sout1):
        wid = lax.axis_index("s") * NC + lax.axis_index("c")
        cg = lax.rem(wid, NCG)
        th = wid // NCG
        h0 = cg * CPG
        n0 = th * tpw
        pltpu.sync_copy(tab_hbm.at[pl.ds(h0 * 2 * KLEV, CPG * 2 * KLEV)], tab_v)

        def in_copy(c, buf, sem):
            return pltpu.make_async_copy(
                x_hbm.at[pl.ds(n0 + c * tn, tn), pl.ds(h0, CPG)], buf, sem)

        def out_copy(c, buf, sem):
            return pltpu.make_async_copy(
                buf, out_hbm.at[pl.ds(n0 + c * tn, tn), pl.ds(h0, CPG)], sem)

        in_copy(0, xb0, sin0).start()
        in_copy(1, xb1, sin1).start()

        lane = lax.iota(jnp.int32, L)
        chbs = [(lane + L * g) * (2 * KLEV) for g in range(ngrp)]
        c16 = jnp.full((L,), 16, jnp.int32)
        c32 = jnp.full((L,), KLEV, jnp.int32)
        t8s = [plsc.load_gather(tab_v, [chb + 8]) for chb in chbs]
        t16s = [plsc.load_gather(tab_v, [chb + 16]) for chb in chbs]
        t24s = [plsc.load_gather(tab_v, [chb + 24]) for chb in chbs]
        zero = jnp.zeros((L,), jnp.int32)

        def compute(xb, ob):
            @plsc.parallel_loop(0, tn, step=1, unroll=2)
            def row(r):
                for g in range(ngrp):
                    xv = xb[r, pl.ds(L * g, L)]
                    m1 = t16s[g] < xv
                    idx = jnp.where(m1, c16, zero)
                    tc = jnp.where(m1, t24s[g], t8s[g])
                    idx = jnp.where(tc < xv, idx + 8, idx)
                    for b in (4, 2, 1):
                        m = idx + b
                        t = plsc.load_gather(tab_v, [chbs[g] + m])
                        idx = jnp.where(t < xv, m, idx)
                    ob[r, pl.ds(L * g, L)] = plsc.load_gather(
                        tab_v, [chbs[g] + idx + c32])

        def pair(i, carry):
            for p, (xb, ob, sin, sout) in enumerate(
                    ((xb0, ob0, sin0, sout0), (xb1, ob1, sin1, sout1))):
                c = 2 * i + p
                in_copy(c, xb, sin).wait()
                pl.when(i > 0)(lambda ob=ob, sout=sout, c=c:
                               out_copy(c, ob, sout).wait())
                compute(xb, ob)
                out_copy(c, ob, sout).start()
                pl.when(i < nchunk // 2 - 1)(
                    lambda xb=xb, sin=sin, c=c: in_copy(c + 2, xb, sin).start())
            return carry

        lax.fori_loop(0, nchunk // 2, pair, 0)
        out_copy(nchunk - 2, ob0, sout0).wait()
        out_copy(nchunk - 1, ob1, sout1).wait()

    return pl.kernel(
        body,
        compiler_params=pltpu.CompilerParams(needs_layout_passes=False),
        out_type=jax.ShapeDtypeStruct((ntok, h), jnp.float32),
        mesh=plsc.VectorSubcoreMesh(
            core_axis_name="c", subcore_axis_name="s",
            num_cores=NC, num_subcores=NS),
        scratch_types=[
            pltpu.VMEM((CPG * 2 * KLEV,), jnp.float32),
            pltpu.VMEM((tn, CPG), jnp.float32),
            pltpu.VMEM((tn, CPG), jnp.float32),
            pltpu.VMEM((tn, CPG), jnp.float32),
            pltpu.VMEM((tn, CPG), jnp.float32),
            pltpu.SemaphoreType.DMA,
            pltpu.SemaphoreType.DMA,
            pltpu.SemaphoreType.DMA,
            pltpu.SemaphoreType.DMA,
        ],
    )


@functools.lru_cache(maxsize=None)
def _get_sc_call(ntok, h, tn):
    return _build_sc_call(ntok, h, tn)


def kernel(x, thresholds, y):
    shape = x.shape
    h = shape[-1]
    x2 = x.reshape(-1, h)
    ntok = x2.shape[0]
    tab = jnp.concatenate(
        [jnp.full((h, 1), -jnp.inf, jnp.float32), thresholds, y],
        axis=1).reshape(-1)
    out = _get_sc_call(ntok, h, 128)(x2, tab)
    return out.reshape(shape)


# speedup vs baseline: 3965.0051x; 1.0404x over previous
import functools

import jax
import jax.numpy as jnp
from jax import lax
from jax.experimental import pallas as pl
from jax.experimental.pallas import tpu as pltpu
from jax.experimental.pallas import tpu_sc as plsc

NC, NS, L = 2, 16, 16
NW = NC * NS
KLEV = 32
CPG = 128
NCG = 16
NTH = NW // NCG


def _build_sc_call(ntok, h, tn):
    tpw = ntok // NTH
    nchunk = tpw // tn
    ngrp = CPG // L

    def body(x_hbm, tab_hbm, out_hbm,
             tab_v, xb0, xb1, ob0, ob1,
             sin0, sin1, sout0, sout1):
        wid = lax.axis_index("s") * NC + lax.axis_index("c")
        cg = lax.rem(wid, NCG)
        th = wid // NCG
        h0 = cg * CPG
        n0 = th * tpw
        pltpu.sync_copy(tab_hbm.at[pl.ds(h0 * 2 * KLEV, CPG * 2 * KLEV)], tab_v)

        def in_copy(c, buf, sem):
            return pltpu.make_async_copy(
                x_hbm.at[pl.ds(n0 + c * tn, tn), pl.ds(h0, CPG)], buf, sem)

        def out_copy(c, buf, sem):
            return pltpu.make_async_copy(
                buf, out_hbm.at[pl.ds(n0 + c * tn, tn), pl.ds(h0, CPG)], sem)

        in_copy(0, xb0, sin0).start()
        in_copy(1, xb1, sin1).start()

        lane = lax.iota(jnp.int32, L)
        chbs = [(lane + L * g) * (2 * KLEV) for g in range(ngrp)]
        c16 = jnp.full((L,), 16, jnp.int32)
        c32 = jnp.full((L,), KLEV, jnp.int32)
        t8s = [plsc.load_gather(tab_v, [chb + 8]) for chb in chbs]
        t16s = [plsc.load_gather(tab_v, [chb + 16]) for chb in chbs]
        t24s = [plsc.load_gather(tab_v, [chb + 24]) for chb in chbs]
        zero = jnp.zeros((L,), jnp.int32)

        def compute(xb, ob):
            @plsc.parallel_loop(0, tn, step=1, unroll=4)
            def row(r):
                for g in range(ngrp):
                    xv = xb[r, pl.ds(L * g, L)]
                    m1 = t16s[g] < xv
                    idx = jnp.where(m1, c16, zero)
                    tc = jnp.where(m1, t24s[g], t8s[g])
                    idx = jnp.where(tc < xv, idx + 8, idx)
                    for b in (4, 2, 1):
                        m = idx + b
                        t = plsc.load_gather(tab_v, [chbs[g] + m])
                        idx = jnp.where(t < xv, m, idx)
                    ob[r, pl.ds(L * g, L)] = plsc.load_gather(
                        tab_v, [chbs[g] + idx + c32])

        def pair(i, carry):
            for p, (xb, ob, sin, sout) in enumerate(
                    ((xb0, ob0, sin0, sout0), (xb1, ob1, sin1, sout1))):
                c = 2 * i + p
                in_copy(c, xb, sin).wait()
                pl.when(i > 0)(lambda ob=ob, sout=sout, c=c:
                               out_copy(c, ob, sout).wait())
                compute(xb, ob)
                out_copy(c, ob, sout).start()
                pl.when(i < nchunk // 2 - 1)(
                    lambda xb=xb, sin=sin, c=c: in_copy(c + 2, xb, sin).start())
            return carry

        lax.fori_loop(0, nchunk // 2, pair, 0)
        out_copy(nchunk - 2, ob0, sout0).wait()
        out_copy(nchunk - 1, ob1, sout1).wait()

    return pl.kernel(
        body,
        compiler_params=pltpu.CompilerParams(needs_layout_passes=False),
        out_type=jax.ShapeDtypeStruct((ntok, h), jnp.float32),
        mesh=plsc.VectorSubcoreMesh(
            core_axis_name="c", subcore_axis_name="s",
            num_cores=NC, num_subcores=NS),
        scratch_types=[
            pltpu.VMEM((CPG * 2 * KLEV,), jnp.float32),
            pltpu.VMEM((tn, CPG), jnp.float32),
            pltpu.VMEM((tn, CPG), jnp.float32),
            pltpu.VMEM((tn, CPG), jnp.float32),
            pltpu.VMEM((tn, CPG), jnp.float32),
            pltpu.SemaphoreType.DMA,
            pltpu.SemaphoreType.DMA,
            pltpu.SemaphoreType.DMA,
            pltpu.SemaphoreType.DMA,
        ],
    )


@functools.lru_cache(maxsize=None)
def _get_sc_call(ntok, h, tn):
    return _build_sc_call(ntok, h, tn)


def kernel(x, thresholds, y):
    shape = x.shape
    h = shape[-1]
    x2 = x.reshape(-1, h)
    ntok = x2.shape[0]
    tab = jnp.concatenate(
        [jnp.full((h, 1), -jnp.inf, jnp.float32), thresholds, y],
        axis=1).reshape(-1)
    out = _get_sc_call(ntok, h, 128)(x2, tab)
    return out.reshape(shape)


# speedup vs baseline: 4138.2063x; 1.0437x over previous
import functools

import jax
import jax.numpy as jnp
from jax import lax
from jax.experimental import pallas as pl
from jax.experimental.pallas import tpu as pltpu
from jax.experimental.pallas import tpu_sc as plsc

NC, NS, L = 2, 16, 16
NW = NC * NS
KLEV = 32
CPG = 128
NCG = 16
NTH = NW // NCG


def _build_sc_call(ntok, h, tn):
    tpw = ntok // NTH
    nchunk = tpw // tn
    ngrp = CPG // L

    def body(x_hbm, tab_hbm, out_hbm,
             tab_v, xb0, xb1, ob0, ob1,
             sin0, sin1, sout0, sout1):
        wid = lax.axis_index("s") * NC + lax.axis_index("c")
        cg = lax.rem(wid, NCG)
        th = wid // NCG
        h0 = cg * CPG
        n0 = th * tpw
        pltpu.sync_copy(tab_hbm.at[pl.ds(h0 * 2 * KLEV, CPG * 2 * KLEV)], tab_v)

        def in_copy(c, buf, sem):
            return pltpu.make_async_copy(
                x_hbm.at[pl.ds(n0 + c * tn, tn), pl.ds(h0, CPG)], buf, sem)

        def out_copy(c, buf, sem):
            return pltpu.make_async_copy(
                buf, out_hbm.at[pl.ds(n0 + c * tn, tn), pl.ds(h0, CPG)], sem)

        in_copy(0, xb0, sin0).start()
        in_copy(1, xb1, sin1).start()

        lane = lax.iota(jnp.int32, L)
        chbs = [(lane + L * g) * (2 * KLEV) for g in range(ngrp)]
        chb16s = [chb + 16 for chb in chbs]
        t8s = [plsc.load_gather(tab_v, [chb + 8]) for chb in chbs]
        t16s = [plsc.load_gather(tab_v, [chb + 16]) for chb in chbs]
        t24s = [plsc.load_gather(tab_v, [chb + 24]) for chb in chbs]

        def compute(xb, ob):
            @plsc.parallel_loop(0, tn, step=1, unroll=4)
            def row(r):
                for g in range(ngrp):
                    xv = xb[r, pl.ds(L * g, L)]
                    m1 = t16s[g] < xv
                    fidx = jnp.where(m1, chb16s[g], chbs[g])
                    tc = jnp.where(m1, t24s[g], t8s[g])
                    fidx = jnp.where(tc < xv, fidx + 8, fidx)
                    for b in (4, 2, 1):
                        m = fidx + b
                        t = plsc.load_gather(tab_v, [m])
                        fidx = jnp.where(t < xv, m, fidx)
                    ob[r, pl.ds(L * g, L)] = plsc.load_gather(
                        tab_v, [fidx + KLEV])

        def pair(i, carry):
            for p, (xb, ob, sin, sout) in enumerate(
                    ((xb0, ob0, sin0, sout0), (xb1, ob1, sin1, sout1))):
                c = 2 * i + p
                in_copy(c, xb, sin).wait()
                pl.when(i > 0)(lambda ob=ob, sout=sout, c=c:
                               out_copy(c, ob, sout).wait())
                compute(xb, ob)
                out_copy(c, ob, sout).start()
                pl.when(i < nchunk // 2 - 1)(
                    lambda xb=xb, sin=sin, c=c: in_copy(c + 2, xb, sin).start())
            return carry

        lax.fori_loop(0, nchunk // 2, pair, 0)
        out_copy(nchunk - 2, ob0, sout0).wait()
        out_copy(nchunk - 1, ob1, sout1).wait()

    return pl.kernel(
        body,
        compiler_params=pltpu.CompilerParams(needs_layout_passes=False),
        out_type=jax.ShapeDtypeStruct((ntok, h), jnp.float32),
        mesh=plsc.VectorSubcoreMesh(
            core_axis_name="c", subcore_axis_name="s",
            num_cores=NC, num_subcores=NS),
        scratch_types=[
            pltpu.VMEM((CPG * 2 * KLEV,), jnp.float32),
            pltpu.VMEM((tn, CPG), jnp.float32),
            pltpu.VMEM((tn, CPG), jnp.float32),
            pltpu.VMEM((tn, CPG), jnp.float32),
            pltpu.VMEM((tn, CPG), jnp.float32),
            pltpu.SemaphoreType.DMA,
            pltpu.SemaphoreType.DMA,
            pltpu.SemaphoreType.DMA,
            pltpu.SemaphoreType.DMA,
        ],
    )


@functools.lru_cache(maxsize=None)
def _get_sc_call(ntok, h, tn):
    return _build_sc_call(ntok, h, tn)


def kernel(x, thresholds, y):
    shape = x.shape
    h = shape[-1]
    x2 = x.reshape(-1, h)
    ntok = x2.shape[0]
    tab = jnp.concatenate(
        [jnp.full((h, 1), -jnp.inf, jnp.float32), thresholds, y],
        axis=1).reshape(-1)
    out = _get_sc_call(ntok, h, 128)(x2, tab)
    return out.reshape(shape)
